# Initial kernel scaffold; baseline (speedup 1.0000x reference)
import jax
import jax.numpy as jnp
from jax.experimental import pallas as pl


def kernel(x, edge_index, edge_attr, pos, batch, W1, W2, g1, b1, g2, b2, Wfc):
    raise NotImplementedError("write your pallas kernel here")


# speedup vs baseline: 1.0000x; 1.0000x over previous
import jax
import jax.numpy as jnp
from jax.experimental import pallas as pl

N = 10000
E = 640000
DIM = 3
K = 3
B = 16
SIZE = 16
NUM_OUT = 101


def _spline_conv(x, src, dst, pseudo, W):
    n = x.shape[0]
    xw = jnp.einsum('ni,kio->nko', x, W)
    v = pseudo * (K - 1)
    bot = jnp.floor(v)
    frac = v - bot
    boti = bot.astype(jnp.int32)
    msg = jnp.zeros((src.shape[0], W.shape[2]), x.dtype)
    for c in range(2 ** DIM):
        b = jnp.ones((pseudo.shape[0],), x.dtype)
        idx = jnp.zeros((pseudo.shape[0],), jnp.int32)
        stride = 1
        for d in range(DIM):
            bit = (c >> d) & 1
            i_d = jnp.clip(boti[:, d] + bit, 0, K - 1)
            b = b * (frac[:, d] if bit else (1.0 - frac[:, d]))
            idx = idx + i_d * stride
            stride = stride * K
        msg = msg + b[:, None] * xw[src, idx]
    deg = jax.ops.segment_sum(jnp.ones((src.shape[0],), x.dtype), dst, num_segments=n)
    agg = jax.ops.segment_sum(msg, dst, num_segments=n)
    return agg / jnp.clip(deg, 1.0)[:, None]


def _batch_norm(x, g, b):
    mu = jnp.mean(x, axis=0)
    var = jnp.var(x, axis=0)
    return (x - mu) / jnp.sqrt(var + 1e-5) * g + b


def _max_pool_x(x, pos2, batch):
    cx = jnp.clip((pos2[:, 0] * 4.0).astype(jnp.int32), 0, 3)
    cy = jnp.clip((pos2[:, 1] * 4.0).astype(jnp.int32), 0, 3)
    seg = batch.astype(jnp.int32) * SIZE + cx + 4 * cy
    out = jax.ops.segment_max(x, seg, num_segments=B * SIZE)
    return jnp.where(jnp.isfinite(out), out, 0.0)


def _id_kernel(x_ref, o_ref):
    o_ref[...] = x_ref[...]


def kernel(x, edge_index, edge_attr, pos, batch, W1, W2, g1, b1, g2, b2, Wfc):
    src, dst = edge_index[0], edge_index[1]
    h = jax.nn.elu(_spline_conv(x, src, dst, edge_attr, W1))
    h = _batch_norm(h, g1, b1)
    h = jax.nn.elu(_spline_conv(h, src, dst, edge_attr, W2))
    h = _batch_norm(h, g2, b2)
    p = _max_pool_x(h, pos[:, :2], batch)
    p = p.reshape(-1, 128 * SIZE)
    out = p @ Wfc
    out = pl.pallas_call(
        _id_kernel,
        out_shape=jax.ShapeDtypeStruct(out.shape, out.dtype),
    )(out)
    return out


# speedup vs baseline: 15.4948x; 15.4946x over previous
import functools

import jax
import jax.numpy as jnp
from jax import lax
from jax.experimental import pallas as pl
from jax.experimental.pallas import tpu as pltpu
from jax.experimental.pallas import tpu_sc as plsc

N = 10000
E = 640000
DIM = 3
K = 3
B = 16
SIZE = 16
NUM_OUT = 101
F1 = 32
F2 = 128

NC = 2
NS = 16
NW = NC * NS
EPW = E // NW
BLK = 400
CH = 80
GRP = 16
NBLK = EPW // BLK
NCH = BLK // CH
NGRP = CH // GRP
NP = 10240
RPT = NP // NS

_CORNERS = [(c & 1, (c >> 1) & 1, (c >> 2) & 1) for c in range(8)]


def _basis(ea0, ea1, ea2):
    one = jnp.float32(1.0)
    zero = jnp.float32(0.0)
    fracs = []
    ibase = None
    for d, ea in enumerate((ea0, ea1, ea2)):
        v = ea * jnp.float32(K - 1)
        bot = jnp.where(v >= one, one, zero)
        fracs.append(v - bot)
        ib = bot.astype(jnp.int32) * jnp.int32(3 ** d)
        ibase = ib if ibase is None else ibase + ib
    return ibase, fracs


def _corner_w(fracs, bits):
    w = None
    for d in range(DIM):
        f = fracs[d] if bits[d] else jnp.float32(1.0) - fracs[d]
        w = f if w is None else w * f
    return w


def _corner_off(bits):
    return jnp.int32(bits[0] + 3 * bits[1] + 9 * bits[2])


_GDN = lax.GatherDimensionNumbers(
    offset_dims=(), collapsed_slice_dims=(0,), start_index_map=(0,))


def _lane_bcast(vec, jfull):
    return lax.gather(vec, jfull[:, None], _GDN, (1,),
                      mode=lax.GatherScatterMode.PROMISE_IN_BOUNDS)


def _sc1_body(src3_h, dst3_h, ea0_h, ea1_h, ea2_h, x_h, z_h, out_h,
              src3_v, xg_v, ea0_v, ea1_v, ea2_v, dst3_v, rowbuf, acc_sh):
    cid = lax.axis_index("c")
    sid = lax.axis_index("s")
    wid = cid * NS + sid
    iota = lax.iota(jnp.int32, 16)
    iota_f = iota.astype(jnp.float32)
    iota_hi_f = iota_f + jnp.float32(16.0)
    deg_hi = jnp.where(iota + 16 == 27, 1.0, 0.0).astype(jnp.float32)

    pltpu.sync_copy(z_h.at[pl.ds(sid * RPT, RPT)],
                    acc_sh.at[pl.ds(sid * RPT, RPT)])

    z16 = jnp.zeros((16,), jnp.float32)

    def zrow(i, _):
        rowbuf[i // 8, pl.ds((i % 8) * 16, 16)] = z16
        return 0
    lax.fori_loop(0, CH * 8, zrow, 0)
    plsc.subcore_barrier()

    def block_body(blk, _):
        base = wid * EPW + blk * BLK
        pltpu.sync_copy(ea0_h.at[pl.ds(base, BLK)], ea0_v)
        pltpu.sync_copy(ea1_h.at[pl.ds(base, BLK)], ea1_v)
        pltpu.sync_copy(ea2_h.at[pl.ds(base, BLK)], ea2_v)
        pltpu.sync_copy(src3_h.at[wid * NBLK + blk], src3_v)
        pltpu.sync_copy(dst3_h.at[wid * NBLK + blk], dst3_v)

        def chunk_body(ci, _):
            pltpu.sync_copy(x_h.at[src3_v.at[ci]], xg_v)

            def group_body(g, _):
                e0 = ci * CH + g * GRP
                ibase, fracs = _basis(ea0_v[pl.ds(e0, 16)],
                                      ea1_v[pl.ds(e0, 16)],
                                      ea2_v[pl.ds(e0, 16)])
                ws = [_corner_w(fracs, bits) for bits in _CORNERS]
                idxs = [(ibase + _corner_off(bits)).astype(jnp.float32)
                        for bits in _CORNERS]

                def j_body(j, _):
                    jf = jnp.full((16,), j, jnp.int32)
                    lo = jnp.zeros((16,), jnp.float32)
                    hi = jnp.zeros((16,), jnp.float32)
                    for c in range(8):
                        wb = _lane_bcast(ws[c], jf)
                        ib = _lane_bcast(idxs[c], jf)
                        lo = lo + jnp.where(iota_f == ib, wb, 0.0)
                        hi = hi + jnp.where(iota_hi_f == ib, wb, 0.0)
                    xrow = xg_v[g * GRP + j, pl.ds(0, 16)]
                    rowbuf[g * GRP + j, pl.ds(0, 16)] = lo * xrow
                    rowbuf[g * GRP + j, pl.ds(16, 16)] = hi * xrow + deg_hi
                    return 0
                lax.fori_loop(0, GRP, j_body, 0)
                return 0
            lax.fori_loop(0, NGRP, group_body, 0)
            pltpu.sync_copy(rowbuf, acc_sh.at[dst3_v.at[ci]], add=True)
            return 0
        lax.fori_loop(0, NCH, chunk_body, 0)
        return 0
    lax.fori_loop(0, NBLK, block_body, 0)

    plsc.subcore_barrier()
    pltpu.sync_copy(acc_sh.at[pl.ds(sid * RPT, RPT)],
                    out_h.at[cid, pl.ds(sid * RPT, RPT)])


def _sc_spline1(src3, dst3, ea0, ea1, ea2, xf, z32):
    mesh = plsc.VectorSubcoreMesh(core_axis_name="c", subcore_axis_name="s", num_cores=NC, num_subcores=NS)
    run = pl.kernel(
        _sc1_body,
        out_type=jax.ShapeDtypeStruct((NC, NP, F2), jnp.float32),
        mesh=mesh,
        compiler_params=pltpu.CompilerParams(needs_layout_passes=False),
        scratch_types=[
            pltpu.VMEM((NCH, CH), jnp.int32),
            pltpu.VMEM((CH, F2), jnp.float32),
            pltpu.VMEM((BLK,), jnp.float32),
            pltpu.VMEM((BLK,), jnp.float32),
            pltpu.VMEM((BLK,), jnp.float32),
            pltpu.VMEM((NCH, CH), jnp.int32),
            pltpu.VMEM((CH, F2), jnp.float32),
            pltpu.VMEM_SHARED((NP, F2), jnp.float32),
        ],
    )
    return run(src3, dst3, ea0, ea1, ea2, xf, z32)


def _sc2_body(src_h, dst3_h, ea0_h, ea1_h, ea2_h, tab_h, z_h, out_h,
              src_v, ea0_v, ea1_v, ea2_v, dst3_v, gidx_v, rows_v, msg_v,
              acc_sh):
    cid = lax.axis_index("c")
    sid = lax.axis_index("s")
    wid = cid * NS + sid
    iota = lax.iota(jnp.int32, 16)

    pltpu.sync_copy(z_h.at[pl.ds(sid * RPT, RPT)],
                    acc_sh.at[pl.ds(sid * RPT, RPT)])
    plsc.subcore_barrier()

    def block_body(blk, _):
        base = wid * EPW + blk * BLK
        pltpu.sync_copy(src_h.at[pl.ds(base, BLK)], src_v)
        pltpu.sync_copy(ea0_h.at[pl.ds(base, BLK)], ea0_v)
        pltpu.sync_copy(ea1_h.at[pl.ds(base, BLK)], ea1_v)
        pltpu.sync_copy(ea2_h.at[pl.ds(base, BLK)], ea2_v)
        pltpu.sync_copy(dst3_h.at[wid * NBLK + blk], dst3_v)

        def chunk_body(ci, _):
            def group_body(g, _):
                e0 = ci * CH + g * GRP
                src_g = src_v[pl.ds(e0, 16)]
                ibase, fracs = _basis(ea0_v[pl.ds(e0, 16)],
                                      ea1_v[pl.ds(e0, 16)],
                                      ea2_v[pl.ds(e0, 16)])
                src27 = src_g * jnp.int32(27)
                bws = []
                for c, bits in enumerate(_CORNERS):
                    plsc.store_scatter(
                        gidx_v, [iota * 8 + jnp.int32(c)],
                        src27 + ibase + _corner_off(bits))
                    bws.append(_corner_w(fracs, bits))
                pltpu.sync_copy(tab_h.at[gidx_v], rows_v)

                def j_body(j, _):
                    jfull = jnp.full((16,), j, jnp.int32)
                    row = j * 8
                    for f in range(8):
                        acc = jnp.zeros((16,), jnp.float32)
                        for c in range(8):
                            w = _lane_bcast(bws[c], jfull)
                            acc = acc + w * rows_v[row + c, pl.ds(f * 16, 16)]
                        msg_v[g * GRP + j, pl.ds(f * 16, 16)] = acc
                    return 0
                lax.fori_loop(0, GRP, j_body, 0)
                return 0
            lax.fori_loop(0, NGRP, group_body, 0)
            pltpu.sync_copy(msg_v, acc_sh.at[dst3_v.at[ci]], add=True)
            return 0
        lax.fori_loop(0, NCH, chunk_body, 0)
        return 0
    lax.fori_loop(0, NBLK, block_body, 0)

    plsc.subcore_barrier()
    pltpu.sync_copy(acc_sh.at[pl.ds(sid * RPT, RPT)],
                    out_h.at[cid, pl.ds(sid * RPT, RPT)])


def _sc_spline2(src, dst3, ea0, ea1, ea2, tab, z128):
    mesh = plsc.VectorSubcoreMesh(core_axis_name="c", subcore_axis_name="s", num_cores=NC, num_subcores=NS)
    run = pl.kernel(
        _sc2_body,
        out_type=jax.ShapeDtypeStruct((NC, NP, F2), jnp.float32),
        mesh=mesh,
        compiler_params=pltpu.CompilerParams(needs_layout_passes=False),
        scratch_types=[
            pltpu.VMEM((BLK,), jnp.int32),
            pltpu.VMEM((BLK,), jnp.float32),
            pltpu.VMEM((BLK,), jnp.float32),
            pltpu.VMEM((BLK,), jnp.float32),
            pltpu.VMEM((NCH, CH), jnp.int32),
            pltpu.VMEM((GRP * 8,), jnp.int32),
            pltpu.VMEM((GRP * 8, F2), jnp.float32),
            pltpu.VMEM((CH, F2), jnp.float32),
            pltpu.VMEM_SHARED((NP, F2), jnp.float32),
        ],
    )
    return run(src, dst3, ea0, ea1, ea2, tab, z128)


def _elu(x):
    return jnp.where(x > 0, x, jnp.exp(x) - 1.0)


def _bn(h, g, b):
    mu = jnp.mean(h, axis=0, keepdims=True)
    var = jnp.mean((h - mu) * (h - mu), axis=0, keepdims=True)
    return (h - mu) * jax.lax.rsqrt(var + 1e-5) * g + b


def _tc1_body(acc_ref, w1_ref, g_ref, b_ref, h1_ref, deg_ref):
    s = acc_ref[0, :N, :F1] + acc_ref[1, :N, :F1]
    deg = jnp.maximum(s[:, 27:28], 1.0)
    agg = jnp.dot(s[:, :27], w1_ref[...],
                  preferred_element_type=jnp.float32) / deg
    h1_ref[...] = _bn(_elu(agg), g_ref[...], b_ref[...])
    deg_ref[...] = deg


def _tc_dense1(acc1, w1r, g1, b1):
    return pl.pallas_call(
        _tc1_body,
        out_shape=(jax.ShapeDtypeStruct((N, F1), jnp.float32),
                   jax.ShapeDtypeStruct((N, 1), jnp.float32)),
    )(acc1, w1r, g1, b1)


def _tc_table_body(h1_ref, w2_ref, out_ref):
    out_ref[...] = jnp.dot(h1_ref[...], w2_ref[...],
                           preferred_element_type=jnp.float32)


def _tc_table(h1, w2f):
    blk = 400
    return pl.pallas_call(
        _tc_table_body,
        grid=(N // blk,),
        in_specs=[pl.BlockSpec((blk, F1), lambda i: (i, 0)),
                  pl.BlockSpec((F1, 27 * F2), lambda i: (0, 0))],
        out_specs=pl.BlockSpec((blk, 27 * F2), lambda i: (i, 0)),
        out_shape=jax.ShapeDtypeStruct((N, 27 * F2), jnp.float32),
    )(h1, w2f)


def _tc2_body(acc_ref, deg_ref, g_ref, b_ref, h2_ref):
    agg = (acc_ref[0, :N, :] + acc_ref[1, :N, :]) / deg_ref[...]
    h2_ref[...] = _bn(_elu(agg), g_ref[...], b_ref[...])


def _tc_dense2(acc2, deg, g2, b2):
    return pl.pallas_call(
        _tc2_body,
        out_shape=jax.ShapeDtypeStruct((N, F2), jnp.float32),
    )(acc2, deg, g2, b2)


def _tc_pool_body(h2_ref, pos_ref, batch_ref, pool_ref):
    cx = jnp.clip((pos_ref[:, 0:1] * 4.0).astype(jnp.int32), 0, 3)
    cy = jnp.clip((pos_ref[:, 1:2] * 4.0).astype(jnp.int32), 0, 3)
    seg = batch_ref[...] * SIZE + cx + 4 * cy
    h2 = h2_ref[...]
    neg = jnp.float32(-jnp.inf)

    def seg_body(s8, _):
        ms = [jnp.max(jnp.where(seg == s8 * 8 + k, h2, neg), axis=0,
                      keepdims=True)
              for k in range(8)]
        m8 = jnp.concatenate(ms, axis=0)
        pool_ref[pl.ds(pl.multiple_of(s8 * 8, 8), 8), :] = jnp.where(
            jnp.isfinite(m8), m8, 0.0)
        return 0
    lax.fori_loop(0, B * SIZE // 8, seg_body, 0)


def _tc_pool(h2, pos, batch2):
    return pl.pallas_call(
        _tc_pool_body,
        out_shape=jax.ShapeDtypeStruct((B * SIZE, F2), jnp.float32),
    )(h2, pos, batch2)


def _tc_fc_body(p_ref, wfc_ref, out_ref):
    out_ref[...] = jnp.dot(p_ref[...], wfc_ref[...],
                           preferred_element_type=jnp.float32)


def _tc_fc(p2, wfc):
    return pl.pallas_call(
        _tc_fc_body,
        out_shape=jax.ShapeDtypeStruct((B, NUM_OUT), jnp.float32),
    )(p2, wfc)


@jax.jit
def kernel(x, edge_index, edge_attr, pos, batch, W1, W2, g1, b1, g2, b2, Wfc):
    src = edge_index[0]
    dst = edge_index[1]
    ea0 = edge_attr[:, 0]
    ea1 = edge_attr[:, 1]
    ea2 = edge_attr[:, 2]
    dst3 = dst.reshape(NW * NBLK, NCH, CH)
    xf = x[:, 0]
    z128 = jnp.zeros((NP, F2), jnp.float32)

    src3 = src.reshape(NW * NBLK, NCH, CH)
    x128 = jnp.broadcast_to(x, (N, F2))
    acc1 = _sc_spline1(src3, dst3, ea0, ea1, ea2, x128, z128)
    h1, deg = _tc_dense1(acc1, W1[:, 0, :], g1.reshape(1, F1),
                         b1.reshape(1, F1))
    w2f = W2.transpose(1, 0, 2).reshape(F1, 27 * F2)
    tab = _tc_table(h1, w2f).reshape(N * 27, F2)
    acc2 = _sc_spline2(src, dst3, ea0, ea1, ea2, tab, z128)
    h2 = _tc_dense2(acc2, deg, g2.reshape(1, F2), b2.reshape(1, F2))
    p = _tc_pool(h2, pos, batch.reshape(N, 1))
    return _tc_fc(p.reshape(B, SIZE * F2), Wfc)
